# baseline (speedup 1.0000x reference)
import jax
import jax.numpy as jnp
from jax.experimental import pallas as pl
from jax.experimental.pallas import tpu as pltpu

BT = 1024
E = 16
D = 4096
N = 8192
HALF = N // 2


def kernel(x, W):
    xf = x.reshape(-1, x.shape[-1])
    n = xf.shape[0]
    nblk = n // BT

    def body(x_hbm, w_hbm, weights_hbm, topw_hbm, tope_hbm,
             ow, otw, ote, cnt, sems):
        c = jax.lax.axis_index("core")
        cnt[0] = 0

        def step(x_ref, w_ref):
            k = cnt[0]
            xb = x_ref[...]
            w = w_ref[...]
            logits = jax.lax.dot_general(
                xb, w, (((1,), (1,)), ((), ())),
                preferred_element_type=jnp.float32)

            m1 = jnp.max(logits, axis=-1, keepdims=True)
            s = jnp.exp(logits - m1)
            denom = jnp.sum(s, axis=-1, keepdims=True)
            weights = s / denom
            ow[pl.ds(k * BT, BT), :] = weights

            iota = jax.lax.broadcasted_iota(jnp.int32, weights.shape, 1)
            w1 = jnp.max(weights, axis=-1, keepdims=True)
            a1 = jnp.min(jnp.where(weights == w1, iota, E),
                         axis=-1, keepdims=True)
            masked = jnp.where(iota == a1, -jnp.inf, weights)
            w2 = jnp.max(masked, axis=-1, keepdims=True)
            a2 = jnp.min(jnp.where(masked == w2, iota, E),
                         axis=-1, keepdims=True)

            norm = w1 + w2
            otw[pl.ds(k * BT, BT), :] = jnp.concatenate(
                [w1 / norm, w2 / norm], axis=-1)
            ote[pl.ds(k * BT, BT), :] = jnp.concatenate([a1, a2], axis=-1)
            cnt[0] = k + 1

        pipeline = pltpu.emit_pipeline(
            step,
            grid=(nblk,),
            in_specs=[
                pl.BlockSpec((BT, D), lambda i: (i, 0)),
                pl.BlockSpec((E, D), lambda i: (0, 0)),
            ],
            out_specs=[],
            core_axis_name="core",
            dimension_semantics=(pltpu.PARALLEL,),
        )
        pipeline(x_hbm, w_hbm)

        base = c * HALF
        cp0 = pltpu.make_async_copy(
            ow, weights_hbm.at[pl.ds(base, HALF), :], sems.at[0])
        cp1 = pltpu.make_async_copy(
            otw, topw_hbm.at[pl.ds(base, HALF), :], sems.at[1])
        cp2 = pltpu.make_async_copy(
            ote, tope_hbm.at[pl.ds(base, HALF), :], sems.at[2])
        cp0.start(); cp1.start(); cp2.start()
        cp0.wait(); cp1.wait(); cp2.wait()

    run = pl.kernel(
        body,
        out_type=[
            jax.ShapeDtypeStruct((n, E), jnp.float32),
            jax.ShapeDtypeStruct((n, 2), jnp.float32),
            jax.ShapeDtypeStruct((n, 2), jnp.int32),
        ],
        mesh=pltpu.create_tensorcore_mesh("core", num_cores=2),
        scratch_types=[
            pltpu.VMEM((HALF, E), jnp.float32),
            pltpu.VMEM((HALF, 2), jnp.float32),
            pltpu.VMEM((HALF, 2), jnp.int32),
            pltpu.SMEM((1,), jnp.int32),
            pltpu.SemaphoreType.DMA((3,)),
        ],
    )
    return tuple(run(xf, W))


# speedup vs baseline: 1.1180x; 1.1180x over previous
import jax
import jax.numpy as jnp
from jax.experimental import pallas as pl
from jax.experimental.pallas import tpu as pltpu

BT = 1024
E = 16


def _router_kernel(x_ref, w_ref, weights_ref, topw_ref, tope_ref):
    xb = x_ref[...]
    w = w_ref[...]
    logits = jax.lax.dot_general(
        xb, w, (((1,), (1,)), ((), ())),
        preferred_element_type=jnp.float32)

    m1 = jnp.max(logits, axis=-1, keepdims=True)
    s = jnp.exp(logits - m1)
    denom = jnp.sum(s, axis=-1, keepdims=True)
    weights = s / denom
    weights_ref[...] = weights

    iota = jax.lax.broadcasted_iota(jnp.int32, weights.shape, 1)
    w1 = jnp.max(weights, axis=-1, keepdims=True)
    a1 = jnp.min(jnp.where(weights == w1, iota, E), axis=-1, keepdims=True)
    masked = jnp.where(iota == a1, -jnp.inf, weights)
    w2 = jnp.max(masked, axis=-1, keepdims=True)
    a2 = jnp.min(jnp.where(masked == w2, iota, E), axis=-1, keepdims=True)

    norm = w1 + w2
    topw_ref[...] = jnp.concatenate([w1 / norm, w2 / norm], axis=-1)
    tope_ref[...] = jnp.concatenate([a1, a2], axis=-1)


def kernel(x, W):
    xf = x.reshape(-1, x.shape[-1])
    n = xf.shape[0]
    grid = (n // BT,)
    weights, top_w, top_e = pl.pallas_call(
        _router_kernel,
        grid=grid,
        in_specs=[
            pl.BlockSpec((BT, xf.shape[1]), lambda i: (i, 0)),
            pl.BlockSpec((E, xf.shape[1]), lambda i: (0, 0)),
        ],
        out_specs=[
            pl.BlockSpec((BT, E), lambda i: (i, 0)),
            pl.BlockSpec((BT, 2), lambda i: (i, 0)),
            pl.BlockSpec((BT, 2), lambda i: (i, 0)),
        ],
        out_shape=[
            jax.ShapeDtypeStruct((n, E), jnp.float32),
            jax.ShapeDtypeStruct((n, 2), jnp.float32),
            jax.ShapeDtypeStruct((n, 2), jnp.int32),
        ],
        compiler_params=pltpu.CompilerParams(
            dimension_semantics=("arbitrary",)),
    )(xf, W)
    return weights, top_w, top_e
